# baseline (speedup 1.0000x reference)
import functools

import jax
import jax.numpy as jnp
from jax.experimental import pallas as pl
from jax.experimental.pallas import tpu as pltpu

_B, _CIN, _S = 8, 512, 1024
_L, _K, _EMB = 4, 1024, 256
_TAU = 1.0
_TS = 512
_NST = _S // _TS


def _bf(a):
    return a.astype(jnp.bfloat16)


def _rvq_kernel(x_ref, w_in_ref, in_b_ref, w_out_ref, out_b_ref,
                cb_ref, cn_ref, scn_ref, one_ref,
                xqt_ref, idx_ref, logit_ref,
                closs_ref, qloss_ref):
    step = pl.program_id(0) * pl.num_programs(1) + pl.program_id(1)
    nsteps = pl.num_programs(0) * pl.num_programs(1)

    @pl.when(step == 0)
    def _init():
        closs_ref[0, 0] = 0.0
        qloss_ref[0, 0] = 0.0

    r = x_ref[0]
    acc = jnp.zeros_like(r)
    loss_acc = jnp.float32(0.0)
    idx_cols = []
    for i in range(_L):
        xp = (jnp.dot(_bf(w_in_ref[i]), _bf(r),
                      preferred_element_type=jnp.float32)
              + in_b_ref[i][:, None])
        cnorm = jnp.sqrt(jnp.sum(xp * xp, axis=0, keepdims=True))
        xn = xp / jnp.maximum(cnorm, 1e-12)
        sxn = jnp.sum(xn * xn, axis=0)
        g = jax.lax.dot_general(_bf(xn), _bf(cn_ref[i]),
                                (((0,), (1,)), ((), ())),
                                preferred_element_type=jnp.float32)
        dist = (sxn[:, None] - 2.0 * g) + scn_ref[i][None, :]
        logits = -dist * (1.0 / _TAU)
        logit_ref[:, i, :] = logits
        idx = jnp.argmax(logits, axis=1, keepdims=True)
        idx_cols.append(idx)
        oh = (jax.lax.broadcasted_iota(jnp.int32, (_TS, _K), 1)
              == idx).astype(jnp.float32)
        xq = jax.lax.dot_general(cb_ref[i], oh, (((0,), (1,)), ((), ())),
                                 preferred_element_type=jnp.float32,
                                 precision=jax.lax.Precision.HIGHEST)
        diff = xp - xq
        loss_acc = loss_acc + jnp.sum(diff * diff)
        xq_st = xp + (xq - xp) * one_ref[0, 0]
        out = (jnp.dot(_bf(w_out_ref[i]), _bf(xq_st),
                       preferred_element_type=jnp.float32)
               + out_b_ref[i][:, None])
        acc = acc + out
        r = r - out

    xqt_ref[0] = acc
    idx_ref[0] = jnp.concatenate(idx_cols, axis=1)
    closs_ref[0, 0] += loss_acc
    qloss_ref[0, 0] += loss_acc

    @pl.when(step == nsteps - 1)
    def _finish():
        scale = 1.0 / (_B * _EMB * _S)
        closs_ref[0, 0] *= scale
        qloss_ref[0, 0] *= scale


@functools.partial(jax.jit, static_argnames=())
def kernel(x, in_v, in_g, in_b, out_v, out_g, out_b, codebook):
    ivn = jnp.sqrt(jnp.sum(in_v * in_v, axis=2, keepdims=True))
    w_in = in_v * (in_g[:, :, None] / jnp.maximum(ivn, 1e-12))
    ovn = jnp.sqrt(jnp.sum(out_v * out_v, axis=2, keepdims=True))
    w_out = out_v * (out_g[:, :, None] / jnp.maximum(ovn, 1e-12))
    cbn = jnp.linalg.norm(codebook, axis=2, keepdims=True)
    cn = codebook / jnp.maximum(cbn, 1e-12)
    scn = jnp.sum(cn * cn, axis=2)

    grid = (_B, _NST)
    full = lambda shape: pl.BlockSpec(shape, lambda b, st: (0,) * len(shape))
    out_shapes = (
        jax.ShapeDtypeStruct((_B, _CIN, _S), jnp.float32),
        jax.ShapeDtypeStruct((_B, _S, _L), jnp.int32),
        jax.ShapeDtypeStruct((_B * _S, _L, _K), jnp.float32),
        jax.ShapeDtypeStruct((1, 1), jnp.float32),
        jax.ShapeDtypeStruct((1, 1), jnp.float32),
    )
    out_specs = (
        pl.BlockSpec((1, _CIN, _TS), lambda b, st: (b, 0, st)),
        pl.BlockSpec((1, _TS, _L), lambda b, st: (b, st, 0)),
        pl.BlockSpec((_TS, _L, _K), lambda b, st: (b * _NST + st, 0, 0)),
        pl.BlockSpec(memory_space=pltpu.SMEM),
        pl.BlockSpec(memory_space=pltpu.SMEM),
    )
    in_specs = [
        pl.BlockSpec((1, _CIN, _TS), lambda b, st: (b, 0, st)),
        full((_L, _EMB, _CIN)),
        full((_L, _EMB)),
        full((_L, _CIN, _EMB)),
        full((_L, _CIN)),
        full((_L, _K, _EMB)),
        full((_L, _K, _EMB)),
        full((_L, _K)),
        pl.BlockSpec(memory_space=pltpu.SMEM),
    ]
    xqt, idxs, logits, closs, qloss = pl.pallas_call(
        _rvq_kernel,
        grid=grid,
        in_specs=in_specs,
        out_specs=out_specs,
        out_shape=out_shapes,
        compiler_params=pltpu.CompilerParams(
            dimension_semantics=("arbitrary", "arbitrary"),
        ),
    )(x, w_in, in_b, w_out, out_b, codebook, cn, scn,
      jnp.ones((1, 1), jnp.float32))
    return (xqt, idxs.transpose(0, 2, 1), logits,
            closs[0, 0], qloss[0, 0])


# speedup vs baseline: 1.0104x; 1.0104x over previous
import functools

import jax
import jax.numpy as jnp
from jax.experimental import pallas as pl
from jax.experimental.pallas import tpu as pltpu

_B, _CIN, _S = 8, 512, 1024
_L, _K, _EMB = 4, 1024, 256
_TAU = 1.0
_TS = 512
_NST = _S // _TS


def _bf(a):
    return a.astype(jnp.bfloat16)


def _rvq_kernel(x_ref, w_in_ref, in_b_ref, w_out_ref, out_b_ref,
                cb_ref, cn_ref, scn_ref, one_ref,
                xqt_ref, idx_ref, logit_ref,
                closs_ref, qloss_ref):
    step = pl.program_id(0) * pl.num_programs(1) + pl.program_id(1)
    nsteps = pl.num_programs(0) * pl.num_programs(1)

    @pl.when(step == 0)
    def _init():
        closs_ref[0, 0] = 0.0
        qloss_ref[0, 0] = 0.0

    r = x_ref[0]
    acc = jnp.zeros_like(r)
    loss_acc = jnp.float32(0.0)
    idx_cols = []
    for i in range(_L):
        xp = (jnp.dot(_bf(w_in_ref[i]), _bf(r),
                      preferred_element_type=jnp.float32)
              + in_b_ref[i][:, None])
        cnorm = jnp.sqrt(jnp.sum(xp * xp, axis=0, keepdims=True))
        xn = xp / jnp.maximum(cnorm, 1e-12)
        sxn = jnp.sum(xn * xn, axis=0)
        g = jax.lax.dot_general(_bf(xn), _bf(cn_ref[i]),
                                (((0,), (1,)), ((), ())),
                                preferred_element_type=jnp.float32)
        logits = (2.0 * g - sxn[:, None]) - scn_ref[i][None, :]
        logit_ref[:, i, :] = logits
        idx = jnp.argmax(logits, axis=1, keepdims=True)
        idx_cols.append(idx)
        oh = (jax.lax.broadcasted_iota(jnp.int32, (_TS, _K), 1)
              == idx).astype(jnp.float32)
        xq = jax.lax.dot_general(cb_ref[i], oh, (((0,), (1,)), ((), ())),
                                 preferred_element_type=jnp.float32,
                                 precision=jax.lax.Precision.HIGHEST)
        diff = xp - xq
        loss_acc = loss_acc + jnp.sum(diff * diff)
        xq_st = xp + (xq - xp) * one_ref[0, 0]
        out = (jnp.dot(_bf(w_out_ref[i]), _bf(xq_st),
                       preferred_element_type=jnp.float32)
               + out_b_ref[i][:, None])
        acc = acc + out
        r = r - out

    xqt_ref[0] = acc
    idx_ref[0] = jnp.concatenate(idx_cols, axis=1)
    closs_ref[0, 0] += loss_acc
    qloss_ref[0, 0] += loss_acc

    @pl.when(step == nsteps - 1)
    def _finish():
        scale = 1.0 / (_B * _EMB * _S)
        closs_ref[0, 0] *= scale
        qloss_ref[0, 0] *= scale


@functools.partial(jax.jit, static_argnames=())
def kernel(x, in_v, in_g, in_b, out_v, out_g, out_b, codebook):
    ivn = jnp.sqrt(jnp.sum(in_v * in_v, axis=2, keepdims=True))
    w_in = in_v * (in_g[:, :, None] / jnp.maximum(ivn, 1e-12))
    ovn = jnp.sqrt(jnp.sum(out_v * out_v, axis=2, keepdims=True))
    w_out = out_v * (out_g[:, :, None] / jnp.maximum(ovn, 1e-12))
    cbn = jnp.linalg.norm(codebook, axis=2, keepdims=True)
    cn = codebook / jnp.maximum(cbn, 1e-12)
    scn = jnp.sum(cn * cn, axis=2)

    grid = (_B, _NST)
    full = lambda shape: pl.BlockSpec(shape, lambda b, st: (0,) * len(shape))
    out_shapes = (
        jax.ShapeDtypeStruct((_B, _CIN, _S), jnp.float32),
        jax.ShapeDtypeStruct((_B, _S, _L), jnp.int32),
        jax.ShapeDtypeStruct((_B * _S, _L, _K), jnp.float32),
        jax.ShapeDtypeStruct((1, 1), jnp.float32),
        jax.ShapeDtypeStruct((1, 1), jnp.float32),
    )
    out_specs = (
        pl.BlockSpec((1, _CIN, _TS), lambda b, st: (b, 0, st)),
        pl.BlockSpec((1, _TS, _L), lambda b, st: (b, st, 0)),
        pl.BlockSpec((_TS, _L, _K), lambda b, st: (b * _NST + st, 0, 0)),
        pl.BlockSpec(memory_space=pltpu.SMEM),
        pl.BlockSpec(memory_space=pltpu.SMEM),
    )
    in_specs = [
        pl.BlockSpec((1, _CIN, _TS), lambda b, st: (b, 0, st)),
        full((_L, _EMB, _CIN)),
        full((_L, _EMB)),
        full((_L, _CIN, _EMB)),
        full((_L, _CIN)),
        full((_L, _K, _EMB)),
        full((_L, _K, _EMB)),
        full((_L, _K)),
        pl.BlockSpec(memory_space=pltpu.SMEM),
    ]
    xqt, idxs, logits, closs, qloss = pl.pallas_call(
        _rvq_kernel,
        grid=grid,
        in_specs=in_specs,
        out_specs=out_specs,
        out_shape=out_shapes,
        compiler_params=pltpu.CompilerParams(
            dimension_semantics=("arbitrary", "arbitrary"),
        ),
    )(x, w_in, in_b, w_out, out_b, codebook, cn, scn,
      jnp.ones((1, 1), jnp.float32))
    return (xqt, idxs.transpose(0, 2, 1), logits,
            closs[0, 0], qloss[0, 0])
